# baseline (speedup 1.0000x reference)
import functools

import jax
import jax.numpy as jnp
from jax import lax
from jax.experimental import pallas as pl
from jax.experimental.pallas import tpu as pltpu
from jax.experimental.pallas import tpu_sc as plsc


def kernel(h, r, t, ent_embeddings, rel_embeddings):
    B = h.shape[0]
    D = ent_embeddings.shape[1]
    info = plsc.get_sparse_core_info()
    NC, NS = info.num_cores, info.num_subcores
    NW = NC * NS
    b_per_w = B // NW

    mesh = plsc.VectorSubcoreMesh(core_axis_name="c", subcore_axis_name="s")
    out_t = jax.ShapeDtypeStruct((B, D), jnp.float32)

    @functools.partial(
        pl.kernel,
        mesh=mesh,
        out_type=[out_t, out_t, out_t],
        compiler_params=pltpu.CompilerParams(use_tc_tiling_on_sc=False),
        scratch_types=[
            pltpu.VMEM((b_per_w,), jnp.int32),
            pltpu.VMEM((b_per_w,), jnp.int32),
            pltpu.VMEM((b_per_w,), jnp.int32),
            pltpu.VMEM((b_per_w, D), jnp.float32),
            pltpu.VMEM((b_per_w, D), jnp.float32),
            pltpu.VMEM((b_per_w, D), jnp.float32),
            pltpu.SemaphoreType.DMA,
            pltpu.SemaphoreType.DMA,
            pltpu.SemaphoreType.DMA,
        ],
    )
    def gather3(h_hbm, r_hbm, t_hbm, ent_hbm, rel_hbm, oh, ot, orr,
                hi_v, ri_v, ti_v, h_rows, t_rows, r_rows,
                sem_h, sem_t, sem_r):
        wid = lax.axis_index("s") * NC + lax.axis_index("c")
        base = wid * b_per_w
        pltpu.sync_copy(h_hbm.at[pl.ds(base, b_per_w)], hi_v)
        pltpu.sync_copy(t_hbm.at[pl.ds(base, b_per_w)], ti_v)
        pltpu.sync_copy(r_hbm.at[pl.ds(base, b_per_w)], ri_v)
        ch = pltpu.async_copy(ent_hbm.at[hi_v], h_rows, sem_h)
        ct = pltpu.async_copy(ent_hbm.at[ti_v], t_rows, sem_t)
        cr = pltpu.async_copy(rel_hbm.at[ri_v], r_rows, sem_r)
        ch.wait()
        pltpu.sync_copy(h_rows, oh.at[pl.ds(base, b_per_w)])
        ct.wait()
        pltpu.sync_copy(t_rows, ot.at[pl.ds(base, b_per_w)])
        cr.wait()
        pltpu.sync_copy(r_rows, orr.at[pl.ds(base, b_per_w)])

    h_e, t_e, r_e = gather3(h, r, t, ent_embeddings, rel_embeddings)
    return (h_e, t_e, r_e)
